# baseline (speedup 1.0000x reference)
import jax
import jax.numpy as jnp
from jax import lax
from jax.experimental import pallas as pl
from jax.experimental.pallas import tpu as pltpu
from jax.experimental.pallas import tpu_sc as plsc

BATCH = 4096
SEQ = 200
D = 64
L = 16
VOCAB = 1000000

NC = 2
NS = 16
NW = NC * NS
ROWS_PER_W = BATCH // NW
NG = SEQ // L
NROWS_GATHERED = (NG + 1) * L
DEPAD_CHUNK = 160
DEPAD_EMIT = 80
N_DEPAD_CHUNKS = VOCAB // DEPAD_CHUNK


def _body(idx_hbm, table_hbm, out_hbm, tscratch, idx_v, idxrow_v, slab_v,
          rows_v, gsem):
    sid = lax.axis_index("s")

    zeros_i = jnp.zeros((L,), jnp.int32)
    for g in range(NG + 1):
        idxrow_v[pl.ds(2 * g * L, L)] = zeros_i
        idxrow_v[pl.ds((2 * g + 1) * L, L)] = zeros_i

    def depad_step(i, carry):
        c = i * NS + sid

        @pl.when(c < N_DEPAD_CHUNKS)
        def _go():
            r = pl.multiple_of(c * DEPAD_CHUNK, 8)
            pltpu.sync_copy(table_hbm.at[pl.ds(r, DEPAD_CHUNK)], slab_v)
            pltpu.sync_copy(slab_v, tscratch.at[pl.ds(r, DEPAD_CHUNK)])

        return carry

    lax.fori_loop(0, (N_DEPAD_CHUNKS + NS - 1) // NS, depad_step, 0)
    plsc.subcore_barrier()

    wid = sid * NC + lax.axis_index("c")
    row0 = wid * ROWS_PER_W
    lanes = lax.iota(jnp.int32, L)

    def block_step(i, carry):
        b8 = pl.multiple_of(row0 + i * 8, 8)
        pltpu.sync_copy(idx_hbm.at[pl.ds(b8, 8)], idx_v)
        for j in range(8):
            half = lax.shift_right_logical(lanes, 1)
            acc = jnp.full((L,), jnp.iinfo(jnp.int32).max, jnp.int32)
            for g in range(NG):
                iv = idx_v[j, pl.ds(g * L, L)]
                iv2 = iv * 2
                idxrow_v[pl.ds(2 * g * L, L)] = jnp.take(
                    iv2, half, indices_are_sorted=True)
                idxrow_v[pl.ds((2 * g + 1) * L, L)] = jnp.take(
                    iv2, 8 + half, indices_are_sorted=True)
                acc = jnp.minimum(acc, iv)
            iv_t = idx_v[j, pl.ds(SEQ - L, L)]
            idxrow_v[pl.ds(2 * NG * L, L)] = jnp.take(
                iv_t * 2, 8 + half, indices_are_sorted=True)
            idxrow_v[pl.ds((2 * NG + 1) * L, L)] = jnp.zeros((L,), jnp.int32)
            acc = jnp.minimum(acc, iv_t)

            gather = pltpu.async_copy(tscratch.at[idxrow_v], rows_v, gsem)

            row_min = acc[0]
            for g in range(1, L):
                row_min = jnp.minimum(row_min, acc[g])
            gather.wait()

            @pl.when(row_min == 0)
            def _zero_fix():
                zeros_f = jnp.zeros((L,), jnp.float32)

                def group_step(g, carry2):
                    iv = idx_v[j, pl.ds(g * L, L)]
                    for lane in range(L):
                        @pl.when(iv[lane] == 0)
                        def _zero_row(lane=lane):
                            r = g * L + lane
                            for jj in range(D // L):
                                rows_v[r, pl.ds(jj * L, L)] = zeros_f

                    return carry2

                lax.fori_loop(0, NG, group_step, 0)
                for lane in range(8, L):
                    @pl.when(iv_t[lane] == 0)
                    def _zero_tail(lane=lane):
                        r = SEQ - L + lane
                        for jj in range(D // L):
                            rows_v[r, pl.ds(jj * L, L)] = zeros_f

            pltpu.sync_copy(rows_v.at[pl.ds(0, SEQ)], out_hbm.at[b8 + j])
        return carry

    lax.fori_loop(0, ROWS_PER_W // 8, block_step, 0)


def kernel(inputs, embedding_weights):
    mesh = plsc.VectorSubcoreMesh(core_axis_name="c", subcore_axis_name="s")
    return pl.kernel(
        _body,
        out_type=jax.ShapeDtypeStruct((BATCH, SEQ, D), jnp.float32),
        mesh=mesh,
        compiler_params=pltpu.CompilerParams(use_tc_tiling_on_sc=True),
        scratch_types=[
            pltpu.MemorySpace.HBM((VOCAB, D), jnp.float32),
            pltpu.VMEM((8, SEQ), jnp.int32),
            pltpu.VMEM((2 * NROWS_GATHERED,), jnp.int32),
            pltpu.VMEM((DEPAD_CHUNK, D), jnp.float32),
            pltpu.VMEM((NROWS_GATHERED * 2, D), jnp.float32),
            pltpu.SemaphoreType.DMA,
        ],
    )(inputs, embedding_weights)
